# baseline (speedup 1.0000x reference)
import functools
import math

import jax
import jax.numpy as jnp
from jax import lax
from jax.experimental import pallas as pl
from jax.experimental.pallas import tpu as pltpu
from jax.experimental.pallas import tpu_sc as plsc

D_MODEL = 32
SCALE = math.sqrt(float(D_MODEL))

NUM_CORES = 2
NUM_SUBCORES = 16
NW = NUM_CORES * NUM_SUBCORES

B_TOTAL = 16384 * 200
B_PER_W = B_TOTAL // NW
CHUNK = 1600
N_CHUNKS = B_PER_W // CHUNK


def _scale_body(t_ref, o_ref):
    o_ref[...] = t_ref[...] * SCALE


def _scale_table(table):
    return pl.pallas_call(
        _scale_body,
        out_shape=jax.ShapeDtypeStruct(table.shape, table.dtype),
    )(table)


@functools.partial(
    pl.kernel,
    mesh=plsc.VectorSubcoreMesh(core_axis_name="c", subcore_axis_name="s"),
    out_type=jax.ShapeDtypeStruct((B_TOTAL, D_MODEL), jnp.float32),
    scratch_types=[
        pltpu.VMEM((CHUNK,), jnp.int32),
        pltpu.VMEM((CHUNK,), jnp.int32),
        pltpu.VMEM((CHUNK, D_MODEL), jnp.float32),
        pltpu.VMEM((CHUNK, D_MODEL), jnp.float32),
        pltpu.SemaphoreType.DMA,
        pltpu.SemaphoreType.DMA,
        pltpu.SemaphoreType.DMA,
        pltpu.SemaphoreType.DMA,
        pltpu.SemaphoreType.DMA,
        pltpu.SemaphoreType.DMA,
    ],
    compiler_params=pltpu.CompilerParams(use_tc_tiling_on_sc=False),
)
def _gather(table_hbm, idx_hbm, out_hbm,
            idx_v0, idx_v1, rows_v0, rows_v1,
            isem0, isem1, gsem0, gsem1, osem0, osem1):
    idx_v = (idx_v0, idx_v1)
    rows_v = (rows_v0, rows_v1)
    isem = (isem0, isem1)
    gsem = (gsem0, gsem1)
    osem = (osem0, osem1)

    wid = lax.axis_index("s") * NUM_CORES + lax.axis_index("c")
    base = wid * B_PER_W

    def chunk_off(j):
        return pl.multiple_of(base + j * CHUNK, CHUNK)

    def issue_idx(j, b):
        pltpu.async_copy(idx_hbm.at[pl.ds(chunk_off(j), CHUNK)], idx_v[b],
                         isem[b])

    def wait_idx(b):
        pltpu.make_async_copy(idx_hbm.at[pl.ds(0, CHUNK)], idx_v[b],
                              isem[b]).wait()

    def gather(b):
        pltpu.async_copy(table_hbm.at[idx_v[b]], rows_v[b], gsem[b]).wait()

    def issue_out(j, b):
        pltpu.async_copy(rows_v[b], out_hbm.at[pl.ds(chunk_off(j), CHUNK)],
                         osem[b])

    def wait_out(b):
        pltpu.make_async_copy(rows_v[b], out_hbm.at[pl.ds(0, CHUNK)],
                              osem[b]).wait()

    issue_idx(0, 0)
    issue_idx(1, 1)
    for b in range(2):
        wait_idx(b)
        gather(b)
        issue_out(b, b)
        issue_idx(b + 2, b)

    def body(g, _):
        for b in range(2):
            j = 2 * g + b
            wait_idx(b)
            wait_out(b)
            gather(b)
            issue_out(j, b)
            issue_idx(jnp.minimum(j + 2, N_CHUNKS - 1), b)
        return 0

    lax.fori_loop(1, N_CHUNKS // 2, body, 0, unroll=False)

    for b in range(2):
        wait_out(b)
        wait_idx(b)


def kernel(x, table):
    idx = x.reshape(-1).astype(jnp.int32)
    scaled = _scale_table(table)
    out = _gather(scaled, idx)
    return out.reshape(x.shape + (D_MODEL,))


# speedup vs baseline: 1.0161x; 1.0161x over previous
import functools
import math

import jax
import jax.numpy as jnp
from jax import lax
from jax.experimental import pallas as pl
from jax.experimental.pallas import tpu as pltpu
from jax.experimental.pallas import tpu_sc as plsc

VOCAB = 1000
D_MODEL = 32
SCALE = math.sqrt(float(D_MODEL))

NUM_CORES = 2
NUM_SUBCORES = 16
NW = NUM_CORES * NUM_SUBCORES

BATCH = 16384
SEQ = 200
W_TILE = BATCH // NW

R_STAGE = 40
N_STAGE = SEQ // R_STAGE
LANES = 16
N_GROUPS = 128 // LANES


def _scale_body(t_ref, o_ref):
    o_ref[...] = t_ref[...] * SCALE


def _scale_table(table):
    return pl.pallas_call(
        _scale_body,
        out_shape=jax.ShapeDtypeStruct(table.shape, table.dtype),
    )(table)


@functools.partial(
    pl.kernel,
    mesh=plsc.VectorSubcoreMesh(core_axis_name="c", subcore_axis_name="s"),
    out_type=jax.ShapeDtypeStruct((SEQ, D_MODEL, BATCH), jnp.float32),
    scratch_types=[
        pltpu.VMEM((VOCAB * D_MODEL,), jnp.float32),
        pltpu.VMEM((R_STAGE, W_TILE), jnp.int32),
        pltpu.VMEM((R_STAGE, W_TILE), jnp.int32),
        pltpu.VMEM((D_MODEL, 128), jnp.float32),
        pltpu.VMEM((D_MODEL, 128), jnp.float32),
        pltpu.SemaphoreType.DMA,
        pltpu.SemaphoreType.DMA,
        pltpu.SemaphoreType.DMA,
        pltpu.SemaphoreType.DMA,
    ],
    compiler_params=pltpu.CompilerParams(use_tc_tiling_on_sc=True,
                                         needs_layout_passes=False),
)
def _gather_t(table_hbm, xt_hbm, out_hbm,
              table_v, xs0, xs1, oc0, oc1,
              xsem0, xsem1, osem0, osem1):
    xs = (xs0, xs1)
    oc = (oc0, oc1)
    xsem = (xsem0, xsem1)
    osem = (osem0, osem1)

    wid = lax.axis_index("s") * NUM_CORES + lax.axis_index("c")
    col0 = wid * W_TILE

    pltpu.sync_copy(table_hbm, table_v)

    def issue_stage(s, b):
        pltpu.async_copy(
            xt_hbm.at[pl.ds(s * R_STAGE, R_STAGE), pl.ds(col0, W_TILE)],
            xs[b], xsem[b])

    def wait_stage(b):
        pltpu.make_async_copy(
            xt_hbm.at[pl.ds(0, R_STAGE), pl.ds(0, W_TILE)],
            xs[b], xsem[b]).wait()

    def wait_out(slot):
        pltpu.make_async_copy(
            oc[slot], out_hbm.at[0, :, pl.ds(0, 128)], osem[slot]).wait()

    def compute_chunk(r, k, b, slot):
        def group(g, _):
            off = pl.multiple_of(k * 128 + g * LANES, LANES)
            idxv = xs[b][r, pl.ds(off, LANES)]
            addr = idxv * D_MODEL
            for c in range(D_MODEL):
                val = plsc.load_gather(table_v, [addr + c])
                oc[slot][c, pl.ds(g * LANES, LANES)] = val
            return 0
        lax.fori_loop(0, N_GROUPS, group, 0, unroll=False)

    def issue_chunk(i1, k, slot):
        pltpu.async_copy(
            oc[slot],
            out_hbm.at[i1, :, pl.ds(col0 + k * 128, 128)],
            osem[slot])

    issue_stage(0, 0)
    issue_stage(1, 1)

    for s in range(N_STAGE):
        b = s % 2
        wait_stage(b)

        def row_body(r, _, s=s, b=b):
            i1 = s * R_STAGE + r
            for k in range(4):
                slot = k % 2
                if s == 0 and k < 2:
                    @pl.when(r > 0)
                    def _():
                        wait_out(slot)
                else:
                    wait_out(slot)
                compute_chunk(r, k, b, slot)
                issue_chunk(i1, k, slot)
            return 0

        lax.fori_loop(0, R_STAGE, row_body, 0, unroll=False)
        if s + 2 < N_STAGE:
            issue_stage(s + 2, b)

    wait_out(0)
    wait_out(1)


def kernel(x, table):
    xt = x.T.astype(jnp.int32)
    flat = _scale_table(table).reshape(-1)
    out_t = _gather_t(flat, xt)
    return out_t.transpose(2, 0, 1)


# speedup vs baseline: 1.7420x; 1.7144x over previous
import functools
import math

import jax
import jax.numpy as jnp
from jax import lax
from jax.experimental import pallas as pl
from jax.experimental.pallas import tpu as pltpu
from jax.experimental.pallas import tpu_sc as plsc

VOCAB = 1000
D_MODEL = 32
SCALE = math.sqrt(float(D_MODEL))

NUM_CORES = 2
NUM_SUBCORES = 16
NW = NUM_CORES * NUM_SUBCORES

BATCH = 16384
SEQ = 200
W_TILE = BATCH // NW

R_STAGE = 40
N_STAGE = SEQ // R_STAGE
LANES = 16
N_GROUPS = 128 // LANES


def _scale_body(t_ref, o_ref):
    o_ref[...] = t_ref[...] * SCALE


def _scale_table(table):
    return pl.pallas_call(
        _scale_body,
        out_shape=jax.ShapeDtypeStruct(table.shape, table.dtype),
    )(table)


@functools.partial(
    pl.kernel,
    mesh=plsc.VectorSubcoreMesh(core_axis_name="c", subcore_axis_name="s"),
    out_type=jax.ShapeDtypeStruct((SEQ, D_MODEL, BATCH), jnp.float32),
    scratch_types=[
        pltpu.VMEM((VOCAB * D_MODEL,), jnp.float32),
        pltpu.VMEM((R_STAGE, W_TILE), jnp.int32),
        pltpu.VMEM((R_STAGE, W_TILE), jnp.int32),
        pltpu.VMEM((D_MODEL, 128), jnp.float32),
        pltpu.VMEM((D_MODEL, 128), jnp.float32),
        pltpu.SemaphoreType.DMA,
        pltpu.SemaphoreType.DMA,
        pltpu.SemaphoreType.DMA,
        pltpu.SemaphoreType.DMA,
    ],
    compiler_params=pltpu.CompilerParams(use_tc_tiling_on_sc=True,
                                         needs_layout_passes=False),
)
def _gather_t(table_hbm, xt_hbm, out_hbm,
              table_v, xs0, xs1, oc0, oc1,
              xsem0, xsem1, osem0, osem1):
    xs = (xs0, xs1)
    oc = (oc0, oc1)
    xsem = (xsem0, xsem1)
    osem = (osem0, osem1)

    wid = lax.axis_index("s") * NUM_CORES + lax.axis_index("c")
    col0 = wid * W_TILE

    pltpu.sync_copy(table_hbm, table_v)

    def issue_stage(s, b):
        pltpu.async_copy(
            xt_hbm.at[pl.ds(s * R_STAGE, R_STAGE), pl.ds(col0, W_TILE)],
            xs[b], xsem[b])

    def wait_stage(b):
        pltpu.make_async_copy(
            xt_hbm.at[pl.ds(0, R_STAGE), pl.ds(0, W_TILE)],
            xs[b], xsem[b]).wait()

    def wait_out(slot):
        pltpu.make_async_copy(
            oc[slot], out_hbm.at[0, :, pl.ds(0, 128)], osem[slot]).wait()

    def compute_chunk(r, k, b, slot):
        def group(g, _):
            off = pl.multiple_of(k * 128 + g * LANES, LANES)
            idxv = xs[b][r, pl.ds(off, LANES)]
            addr = idxv * D_MODEL
            vals = [plsc.load_gather(table_v, [addr + c])
                    for c in range(D_MODEL)]
            for c in range(D_MODEL):
                oc[slot][c, pl.ds(g * LANES, LANES)] = vals[c]
            return 0
        lax.fori_loop(0, N_GROUPS, group, 0, unroll=False)

    def issue_chunk(i1, k, slot):
        pltpu.async_copy(
            oc[slot],
            out_hbm.at[i1, :, pl.ds(col0 + k * 128, 128)],
            osem[slot])

    issue_stage(0, 0)
    issue_stage(1, 1)

    for s in range(N_STAGE):
        b = s % 2
        wait_stage(b)

        def row_body(r, _, s=s, b=b):
            i1 = s * R_STAGE + r
            for k in range(4):
                slot = k % 2
                if s == 0 and k < 2:
                    @pl.when(r > 0)
                    def _():
                        wait_out(slot)
                else:
                    wait_out(slot)
                compute_chunk(r, k, b, slot)
                issue_chunk(i1, k, slot)
            return 0

        lax.fori_loop(0, R_STAGE, row_body, 0, unroll=False)
        if s + 2 < N_STAGE:
            issue_stage(s + 2, b)

    wait_out(0)
    wait_out(1)


def kernel(x, table):
    xt = x.T.astype(jnp.int32)
    flat = _scale_table(table).reshape(-1)
    out_t = _gather_t(flat, xt)
    return out_t.transpose(2, 0, 1)


# speedup vs baseline: 1.8539x; 1.0642x over previous
import functools
import math

import jax
import jax.numpy as jnp
from jax import lax
from jax.experimental import pallas as pl
from jax.experimental.pallas import tpu as pltpu
from jax.experimental.pallas import tpu_sc as plsc

VOCAB = 1000
D_MODEL = 32
SCALE = math.sqrt(float(D_MODEL))

NUM_CORES = 2
NUM_SUBCORES = 16
NW = NUM_CORES * NUM_SUBCORES

BATCH = 16384
SEQ = 200
W_TILE = BATCH // NW

R_STAGE = 40
N_STAGE = SEQ // R_STAGE
LANES = 16
N_GROUPS = 128 // LANES


def _scale_body(t_ref, o_ref):
    o_ref[...] = t_ref[...] * SCALE


def _scale_table(table):
    return pl.pallas_call(
        _scale_body,
        out_shape=jax.ShapeDtypeStruct(table.shape, table.dtype),
    )(table)


@functools.partial(
    pl.kernel,
    mesh=plsc.VectorSubcoreMesh(core_axis_name="c", subcore_axis_name="s"),
    out_type=jax.ShapeDtypeStruct((SEQ, D_MODEL, BATCH), jnp.float32),
    scratch_types=[
        pltpu.VMEM((VOCAB * D_MODEL,), jnp.float32),
        pltpu.VMEM((R_STAGE, W_TILE), jnp.int32),
        pltpu.VMEM((R_STAGE, W_TILE), jnp.int32),
        pltpu.VMEM((D_MODEL, 128), jnp.float32),
        pltpu.VMEM((D_MODEL, 128), jnp.float32),
        pltpu.SemaphoreType.DMA,
        pltpu.SemaphoreType.DMA,
        pltpu.SemaphoreType.DMA,
        pltpu.SemaphoreType.DMA,
    ],
    compiler_params=pltpu.CompilerParams(use_tc_tiling_on_sc=True,
                                         needs_layout_passes=False),
)
def _gather_t(table_hbm, xt_hbm, out_hbm,
              table_v, xs0, xs1, oc0, oc1,
              xsem0, xsem1, osem0, osem1):
    xs = (xs0, xs1)
    oc = (oc0, oc1)
    xsem = (xsem0, xsem1)
    osem = (osem0, osem1)

    wid = lax.axis_index("s") * NUM_CORES + lax.axis_index("c")
    col0 = wid * W_TILE

    pltpu.sync_copy(table_hbm, table_v)

    def issue_stage(s, b):
        pltpu.async_copy(
            xt_hbm.at[pl.ds(s * R_STAGE, R_STAGE), pl.ds(col0, W_TILE)],
            xs[b], xsem[b])

    def wait_stage(b):
        pltpu.make_async_copy(
            xt_hbm.at[pl.ds(0, R_STAGE), pl.ds(0, W_TILE)],
            xs[b], xsem[b]).wait()

    def wait_out(slot):
        pltpu.make_async_copy(
            oc[slot], out_hbm.at[0, :, pl.ds(0, 128)], osem[slot]).wait()

    def compute_chunk(r, k, b, slot):
        @plsc.parallel_loop(0, N_GROUPS)
        def group(g):
            off = pl.multiple_of(k * 128 + g * LANES, LANES)
            idxv = xs[b][r, pl.ds(off, LANES)]
            addr = idxv * D_MODEL
            vals = [plsc.load_gather(table_v, [addr + c])
                    for c in range(D_MODEL)]
            for c in range(D_MODEL):
                oc[slot][c, pl.ds(g * LANES, LANES)] = vals[c]

    def issue_chunk(i1, k, slot):
        pltpu.async_copy(
            oc[slot],
            out_hbm.at[i1, :, pl.ds(col0 + k * 128, 128)],
            osem[slot])

    issue_stage(0, 0)
    issue_stage(1, 1)

    for s in range(N_STAGE):
        b = s % 2
        wait_stage(b)

        def row_body(r, _, s=s, b=b):
            i1 = s * R_STAGE + r
            for k in range(4):
                slot = k % 2
                if s == 0 and k < 2:
                    @pl.when(r > 0)
                    def _():
                        wait_out(slot)
                else:
                    wait_out(slot)
                compute_chunk(r, k, b, slot)
                issue_chunk(i1, k, slot)
            return 0

        lax.fori_loop(0, R_STAGE, row_body, 0, unroll=False)
        if s + 2 < N_STAGE:
            issue_stage(s + 2, b)

    wait_out(0)
    wait_out(1)


def kernel(x, table):
    xt = x.T.astype(jnp.int32)
    flat = _scale_table(table).reshape(-1)
    out_t = _gather_t(flat, xt)
    return out_t.transpose(2, 0, 1)


# speedup vs baseline: 10.2014x; 5.5028x over previous
import functools
import math

import jax
import jax.numpy as jnp
from jax import lax
from jax.experimental import pallas as pl
from jax.experimental.pallas import tpu as pltpu
from jax.experimental.pallas import tpu_sc as plsc

VOCAB = 1000
D_MODEL = 32
SCALE = math.sqrt(float(D_MODEL))

NUM_CORES = 2
NUM_SUBCORES = 16
NW = NUM_CORES * NUM_SUBCORES

BATCH = 16384
SEQ = 200
W_TILE = BATCH // NW

R_STAGE = 40
N_STAGE = SEQ // R_STAGE
LANES = 16
N_GROUPS = 128 // LANES


def _scale_body(t_ref, o_ref):
    o_ref[...] = t_ref[...].T * SCALE


def _scale_table(table):
    return pl.pallas_call(
        _scale_body,
        out_shape=jax.ShapeDtypeStruct((D_MODEL, VOCAB), table.dtype),
    )(table)


@functools.partial(
    pl.kernel,
    mesh=plsc.VectorSubcoreMesh(core_axis_name="c", subcore_axis_name="s"),
    out_type=jax.ShapeDtypeStruct((SEQ, D_MODEL, BATCH), jnp.float32),
    scratch_types=[
        pltpu.VMEM((VOCAB * D_MODEL,), jnp.float32),
        pltpu.VMEM((R_STAGE, W_TILE), jnp.int32),
        pltpu.VMEM((R_STAGE, W_TILE), jnp.int32),
        pltpu.VMEM((D_MODEL, 128), jnp.float32),
        pltpu.VMEM((D_MODEL, 128), jnp.float32),
        pltpu.SemaphoreType.DMA,
        pltpu.SemaphoreType.DMA,
        pltpu.SemaphoreType.DMA,
        pltpu.SemaphoreType.DMA,
    ],
    compiler_params=pltpu.CompilerParams(use_tc_tiling_on_sc=True,
                                         needs_layout_passes=False),
)
def _gather_t(table_hbm, xt_hbm, out_hbm,
              table_v, xs0, xs1, oc0, oc1,
              xsem0, xsem1, osem0, osem1):
    xs = (xs0, xs1)
    oc = (oc0, oc1)
    xsem = (xsem0, xsem1)
    osem = (osem0, osem1)

    wid = lax.axis_index("s") * NUM_CORES + lax.axis_index("c")
    col0 = wid * W_TILE

    pltpu.sync_copy(table_hbm, table_v)

    def issue_stage(s, b):
        pltpu.async_copy(
            xt_hbm.at[pl.ds(s * R_STAGE, R_STAGE), pl.ds(col0, W_TILE)],
            xs[b], xsem[b])

    def wait_stage(b):
        pltpu.make_async_copy(
            xt_hbm.at[pl.ds(0, R_STAGE), pl.ds(0, W_TILE)],
            xs[b], xsem[b]).wait()

    def wait_out(slot):
        pltpu.make_async_copy(
            oc[slot], out_hbm.at[0, :, pl.ds(0, 128)], osem[slot]).wait()

    def compute_chunk(r, k, b, slot):
        @plsc.parallel_loop(0, N_GROUPS)
        def group(g):
            off = pl.multiple_of(k * 128 + g * LANES, LANES)
            idxv = xs[b][r, pl.ds(off, LANES)]
            vals = [plsc.load_gather(table_v, [idxv + c * VOCAB])
                    for c in range(D_MODEL)]
            for c in range(D_MODEL):
                oc[slot][c, pl.ds(g * LANES, LANES)] = vals[c]

    def issue_chunk(i1, k, slot):
        pltpu.async_copy(
            oc[slot],
            out_hbm.at[i1, :, pl.ds(col0 + k * 128, 128)],
            osem[slot])

    issue_stage(0, 0)
    issue_stage(1, 1)

    for s in range(N_STAGE):
        b = s % 2
        wait_stage(b)

        def row_body(r, _, s=s, b=b):
            i1 = s * R_STAGE + r
            for k in range(4):
                slot = k % 2
                if s == 0 and k < 2:
                    @pl.when(r > 0)
                    def _():
                        wait_out(slot)
                else:
                    wait_out(slot)
                compute_chunk(r, k, b, slot)
                issue_chunk(i1, k, slot)
            return 0

        lax.fori_loop(0, R_STAGE, row_body, 0, unroll=False)
        if s + 2 < N_STAGE:
            issue_stage(s + 2, b)

    wait_out(0)
    wait_out(1)


def kernel(x, table):
    xt = x.T.astype(jnp.int32)
    flat = _scale_table(table).reshape(-1)
    out_t = _gather_t(flat, xt)
    return out_t.transpose(2, 0, 1)


# speedup vs baseline: 11.9754x; 1.1739x over previous
import functools
import math

import jax
import jax.numpy as jnp
from jax import lax
from jax.experimental import pallas as pl
from jax.experimental.pallas import tpu as pltpu
from jax.experimental.pallas import tpu_sc as plsc

VOCAB = 1000
D_MODEL = 32
SCALE = math.sqrt(float(D_MODEL))

NUM_CORES = 2
NUM_SUBCORES = 16
NW = NUM_CORES * NUM_SUBCORES

BATCH = 16384
SEQ = 200
W_TILE = BATCH // NW

R_STAGE = 40
N_STAGE = SEQ // R_STAGE
LANES = 16
CW = 256
N_GROUPS = CW // LANES
K_CHUNKS = W_TILE // CW


def _scale_body(t_ref, o_ref):
    o_ref[...] = t_ref[...].T * SCALE


def _scale_table(table):
    return pl.pallas_call(
        _scale_body,
        out_shape=jax.ShapeDtypeStruct((D_MODEL, VOCAB), table.dtype),
    )(table)


@functools.partial(
    pl.kernel,
    mesh=plsc.VectorSubcoreMesh(core_axis_name="c", subcore_axis_name="s"),
    out_type=jax.ShapeDtypeStruct((SEQ, D_MODEL, BATCH), jnp.float32),
    scratch_types=[
        pltpu.VMEM((VOCAB * D_MODEL,), jnp.float32),
        pltpu.VMEM((R_STAGE, W_TILE), jnp.int32),
        pltpu.VMEM((R_STAGE, W_TILE), jnp.int32),
        pltpu.VMEM((D_MODEL, CW), jnp.float32),
        pltpu.VMEM((D_MODEL, CW), jnp.float32),
        pltpu.SemaphoreType.DMA,
        pltpu.SemaphoreType.DMA,
        pltpu.SemaphoreType.DMA,
        pltpu.SemaphoreType.DMA,
    ],
    compiler_params=pltpu.CompilerParams(use_tc_tiling_on_sc=True,
                                         needs_layout_passes=False),
)
def _gather_t(table_hbm, xt_hbm, out_hbm,
              table_v, xs0, xs1, oc0, oc1,
              xsem0, xsem1, osem0, osem1):
    xs = (xs0, xs1)
    oc = (oc0, oc1)
    xsem = (xsem0, xsem1)
    osem = (osem0, osem1)

    wid = lax.axis_index("s") * NUM_CORES + lax.axis_index("c")
    col0 = wid * W_TILE

    pltpu.sync_copy(table_hbm, table_v)

    def issue_stage(s, b):
        pltpu.async_copy(
            xt_hbm.at[pl.ds(s * R_STAGE, R_STAGE), pl.ds(col0, W_TILE)],
            xs[b], xsem[b])

    def wait_stage(b):
        pltpu.make_async_copy(
            xt_hbm.at[pl.ds(0, R_STAGE), pl.ds(0, W_TILE)],
            xs[b], xsem[b]).wait()

    def wait_out(slot):
        pltpu.make_async_copy(
            oc[slot], out_hbm.at[0, :, pl.ds(0, CW)], osem[slot]).wait()

    def compute_chunk(r, k, b, slot):
        @plsc.parallel_loop(0, N_GROUPS)
        def group(g):
            off = pl.multiple_of(k * CW + g * LANES, LANES)
            idxv = xs[b][r, pl.ds(off, LANES)]
            vals = [plsc.load_gather(table_v, [idxv + c * VOCAB])
                    for c in range(D_MODEL)]
            for c in range(D_MODEL):
                oc[slot][c, pl.ds(g * LANES, LANES)] = vals[c]

    def issue_chunk(i1, k, slot):
        pltpu.async_copy(
            oc[slot],
            out_hbm.at[i1, :, pl.ds(col0 + k * CW, CW)],
            osem[slot])

    issue_stage(0, 0)
    issue_stage(1, 1)

    for s in range(N_STAGE):
        b = s % 2
        wait_stage(b)

        def row_body(r, _, s=s, b=b):
            i1 = s * R_STAGE + r
            for k in range(K_CHUNKS):
                slot = k % 2
                if s == 0 and k < 2:
                    @pl.when(r > 0)
                    def _():
                        wait_out(slot)
                else:
                    wait_out(slot)
                compute_chunk(r, k, b, slot)
                issue_chunk(i1, k, slot)
            return 0

        lax.fori_loop(0, R_STAGE, row_body, 0, unroll=False)
        if s + 2 < N_STAGE:
            issue_stage(s + 2, b)

    wait_out(0)
    wait_out(1)


def kernel(x, table):
    xt = x.T.astype(jnp.int32)
    flat = _scale_table(table).reshape(-1)
    out_t = _gather_t(flat, xt)
    return out_t.transpose(2, 0, 1)
